# Initial kernel scaffold; baseline (speedup 1.0000x reference)
import jax
import jax.numpy as jnp
from jax.experimental import pallas as pl


def kernel(boxes_norm, grid_size, h_table, w_table):
    raise NotImplementedError("write your pallas kernel here")


# speedup vs baseline: 1.8598x; 1.8598x over previous
import dataclasses
import functools

import jax
import jax.numpy as jnp
from jax import lax
from jax.experimental import pallas as pl
from jax.experimental.pallas import tpu as pltpu
from jax.experimental.pallas import tpu_sc as plsc

NC = 2
NS = 16
L = 16
NW = NC * NS

_MAGIC = 8388608.0


def _build_sc_gather(n_boxes, n_rows_h, d):
    rows_total = 2 * n_boxes
    rows_per_w = rows_total // NW
    boxes_per_w = n_boxes // NW
    flat_per_w = 4 * boxes_per_w
    CH = 128
    n_chunks = rows_per_w // CH
    n_vec = rows_per_w // L

    mesh = plsc.VectorSubcoreMesh(core_axis_name="c", subcore_axis_name="s")
    cp = pltpu.CompilerParams()
    if "needs_layout_passes" in pltpu.CompilerParams.__dataclass_fields__:
        cp = dataclasses.replace(cp, needs_layout_passes=False)

    @functools.partial(
        pl.kernel,
        mesh=mesh,
        compiler_params=cp,
        out_type=jax.ShapeDtypeStruct((rows_total, d), jnp.float32),
        scratch_types=[
            pltpu.VMEM((flat_per_w,), jnp.float32),
            pltpu.VMEM((L,), jnp.float32),
            pltpu.VMEM((rows_per_w,), jnp.int32),
            pltpu.VMEM((CH, d), jnp.float32),
            pltpu.VMEM((CH, d), jnp.float32),
            pltpu.SemaphoreType.DMA,
            pltpu.SemaphoreType.DMA,
            pltpu.SemaphoreType.DMA,
            pltpu.SemaphoreType.DMA,
        ],
    )
    def sc_kernel(boxes_hbm, scale_hbm, table_hbm, out_hbm,
                  bx_v, scale_v, idx_v, rows_a, rows_b,
                  gsem_a, gsem_b, osem_a, osem_b):
        wid = lax.axis_index("s") * NC + lax.axis_index("c")
        pltpu.sync_copy(boxes_hbm.at[pl.ds(wid * flat_per_w, flat_per_w)], bx_v)
        pltpu.sync_copy(scale_hbm, scale_v)
        scale = scale_v[...]

        jvec = lax.iota(jnp.int32, L)
        half = lax.shift_right_logical(jvec, 1)
        parity = lax.bitwise_and(jvec, 1)
        fbase = 4 * half + 1 - parity
        off = parity * n_rows_h

        @pl.loop(0, n_vec)
        def _(t):
            fidx = (2 * L) * t + fbase
            v = plsc.load_gather(bx_v, [fidx])
            r = (v * scale + _MAGIC) - _MAGIC
            idx_v[pl.ds(t * L, L)] = r.astype(jnp.int32) + off

        wbase = wid * rows_per_w
        bufs = (rows_a, rows_b)
        gsems = (gsem_a, gsem_b)
        osems = (osem_a, osem_b)
        gd = [None, None]
        od = [None, None]
        gd[0] = pltpu.async_copy(table_hbm.at[idx_v.at[pl.ds(0, CH)]],
                                 bufs[0], gsems[0])
        for c in range(n_chunks):
            b = c % 2
            gd[b].wait()
            if c + 1 < n_chunks:
                gd[1 - b] = pltpu.async_copy(
                    table_hbm.at[idx_v.at[pl.ds((c + 1) * CH, CH)]],
                    bufs[1 - b], gsems[1 - b])
            if od[b] is not None:
                od[b].wait()
            od[b] = pltpu.async_copy(
                bufs[b], out_hbm.at[pl.ds(wbase + c * CH, CH)], osems[b])
        for x in od:
            if x is not None:
                x.wait()

    return sc_kernel


def kernel(boxes_norm, grid_size, h_table, w_table):
    n, _ = boxes_norm.shape
    n_rows_h, d = h_table.shape
    table = jnp.concatenate([h_table, w_table], axis=0)
    scale = jnp.full((L,), (grid_size - 1), dtype=jnp.float32)
    boxes_flat = boxes_norm.reshape(-1)
    sc = _build_sc_gather(n, n_rows_h, d)
    out = sc(boxes_flat, scale, table)
    return out.reshape(n, 2 * d, 1, 1)


# speedup vs baseline: 1.8914x; 1.0170x over previous
import dataclasses
import functools

import jax
import jax.numpy as jnp
from jax import lax
from jax.experimental import pallas as pl
from jax.experimental.pallas import tpu as pltpu
from jax.experimental.pallas import tpu_sc as plsc

NC = 2
NS = 16
L = 16
NW = NC * NS

_MAGIC = 8388608.0


def _build_sc_gather(n_boxes, n_rows_h, d):
    rows_total = 2 * n_boxes
    rows_per_w = rows_total // NW
    boxes_per_w = n_boxes // NW
    flat_per_w = 4 * boxes_per_w
    CH = 256
    NBUF = 3
    n_chunks = rows_per_w // CH
    n_vec = rows_per_w // L

    mesh = plsc.VectorSubcoreMesh(core_axis_name="c", subcore_axis_name="s")
    cp = pltpu.CompilerParams()
    if "needs_layout_passes" in pltpu.CompilerParams.__dataclass_fields__:
        cp = dataclasses.replace(cp, needs_layout_passes=False)

    @functools.partial(
        pl.kernel,
        mesh=mesh,
        compiler_params=cp,
        out_type=jax.ShapeDtypeStruct((rows_total, d), jnp.float32),
        scratch_types=(
            [pltpu.VMEM((flat_per_w,), jnp.float32),
             pltpu.VMEM((L,), jnp.float32),
             pltpu.VMEM((rows_per_w,), jnp.int32)]
            + [pltpu.VMEM((CH, d), jnp.float32) for _ in range(NBUF)]
            + [pltpu.SemaphoreType.DMA for _ in range(2 * NBUF)]
        ),
    )
    def sc_kernel(boxes_hbm, scale_hbm, table_hbm, out_hbm,
                  bx_v, scale_v, idx_v, *bufs_and_sems):
        bufs = bufs_and_sems[:NBUF]
        gsems = bufs_and_sems[NBUF:2 * NBUF]
        osems = bufs_and_sems[2 * NBUF:3 * NBUF]
        wid = lax.axis_index("s") * NC + lax.axis_index("c")
        pltpu.sync_copy(boxes_hbm.at[pl.ds(wid * flat_per_w, flat_per_w)], bx_v)
        pltpu.sync_copy(scale_hbm, scale_v)
        scale = scale_v[...]

        jvec = lax.iota(jnp.int32, L)
        half = lax.shift_right_logical(jvec, 1)
        parity = lax.bitwise_and(jvec, 1)
        fbase = 4 * half + 1 - parity
        off = parity * n_rows_h

        @pl.loop(0, n_vec)
        def _(t):
            fidx = (2 * L) * t + fbase
            v = plsc.load_gather(bx_v, [fidx])
            r = (v * scale + _MAGIC) - _MAGIC
            idx_v[pl.ds(t * L, L)] = r.astype(jnp.int32) + off

        wbase = wid * rows_per_w
        gd = [None] * NBUF
        od = [None] * NBUF

        def start_gather(c):
            b = c % NBUF
            gd[b] = pltpu.async_copy(
                table_hbm.at[idx_v.at[pl.ds(c * CH, CH)]], bufs[b], gsems[b])

        LK = NBUF - 1
        for c in range(min(LK, n_chunks)):
            start_gather(c)
        for c in range(n_chunks):
            b = c % NBUF
            gd[b].wait()
            if od[b] is not None:
                od[b].wait()
            od[b] = pltpu.async_copy(
                bufs[b], out_hbm.at[pl.ds(wbase + c * CH, CH)], osems[b])
            nxt = c + LK
            if nxt < n_chunks:
                bb = nxt % NBUF
                if od[bb] is not None:
                    od[bb].wait()
                    od[bb] = None
                start_gather(nxt)
        for x in od:
            if x is not None:
                x.wait()

    return sc_kernel


def kernel(boxes_norm, grid_size, h_table, w_table):
    n, _ = boxes_norm.shape
    n_rows_h, d = h_table.shape
    table = jnp.concatenate([h_table, w_table], axis=0)
    scale = jnp.full((L,), (grid_size - 1), dtype=jnp.float32)
    boxes_flat = boxes_norm.reshape(-1)
    sc = _build_sc_gather(n, n_rows_h, d)
    out = sc(boxes_flat, scale, table)
    return out.reshape(n, 2 * d, 1, 1)


# speedup vs baseline: 3.0047x; 1.5886x over previous
import dataclasses
import functools

import jax
import jax.numpy as jnp
from jax import lax
from jax.experimental import pallas as pl
from jax.experimental.pallas import tpu as pltpu
from jax.experimental.pallas import tpu_sc as plsc

NC = 2
NS = 16
L = 16
NW = NC * NS

_MAGIC = 8388608.0


def _build_sc_gather(n_boxes, n_rows_h, d):
    rows_total = 2 * n_boxes
    rows_per_w = rows_total // NW
    boxes_per_w = n_boxes // NW
    flat_per_w = 4 * boxes_per_w
    CH = 256
    NBUF = 3
    n_chunks = rows_per_w // CH
    n_vec = rows_per_w // L

    mesh = plsc.VectorSubcoreMesh(core_axis_name="c", subcore_axis_name="s")
    cp = pltpu.CompilerParams()
    if "needs_layout_passes" in pltpu.CompilerParams.__dataclass_fields__:
        cp = dataclasses.replace(cp, needs_layout_passes=False)

    @functools.partial(
        pl.kernel,
        mesh=mesh,
        compiler_params=cp,
        out_type=jax.ShapeDtypeStruct((rows_total, d), jnp.float32),
        scratch_types=(
            [pltpu.VMEM((flat_per_w,), jnp.float32),
             pltpu.VMEM((L,), jnp.float32),
             pltpu.VMEM((rows_per_w,), jnp.int32)]
            + [pltpu.VMEM((CH, d), jnp.float32) for _ in range(NBUF)]
            + [pltpu.SemaphoreType.DMA for _ in range(2 * NBUF)]
        ),
    )
    def sc_kernel(boxes_hbm, scale_hbm, table_hbm, out_hbm,
                  bx_v, scale_v, idx_v, *bufs_and_sems):
        bufs = bufs_and_sems[:NBUF]
        gsems = bufs_and_sems[NBUF:2 * NBUF]
        osems = bufs_and_sems[2 * NBUF:3 * NBUF]
        wid = lax.axis_index("s") * NC + lax.axis_index("c")
        pltpu.sync_copy(boxes_hbm.at[pl.ds(wid * flat_per_w, flat_per_w)], bx_v)
        pltpu.sync_copy(scale_hbm, scale_v)
        scale = scale_v[...]
        tbl_base = wid * (2 * n_rows_h)

        jvec = lax.iota(jnp.int32, L)
        half = lax.shift_right_logical(jvec, 1)
        parity = lax.bitwise_and(jvec, 1)
        fbase = 4 * half + 1 - parity
        off = parity * n_rows_h

        @pl.loop(0, n_vec)
        def _(t):
            fidx = (2 * L) * t + fbase
            v = plsc.load_gather(bx_v, [fidx])
            r = (v * scale + _MAGIC) - _MAGIC
            idx_v[pl.ds(t * L, L)] = r.astype(jnp.int32) + off + tbl_base

        wbase = wid * rows_per_w
        gd = [None] * NBUF
        od = [None] * NBUF

        def start_gather(c):
            b = c % NBUF
            gd[b] = pltpu.async_copy(
                table_hbm.at[idx_v.at[pl.ds(c * CH, CH)]], bufs[b], gsems[b])

        LK = NBUF - 1
        for c in range(min(LK, n_chunks)):
            start_gather(c)
        for c in range(n_chunks):
            b = c % NBUF
            gd[b].wait()
            if od[b] is not None:
                od[b].wait()
            od[b] = pltpu.async_copy(
                bufs[b], out_hbm.at[pl.ds(wbase + c * CH, CH)], osems[b])
            nxt = c + LK
            if nxt < n_chunks:
                bb = nxt % NBUF
                if od[bb] is not None:
                    od[bb].wait()
                    od[bb] = None
                start_gather(nxt)
        for x in od:
            if x is not None:
                x.wait()

    return sc_kernel


def kernel(boxes_norm, grid_size, h_table, w_table):
    n, _ = boxes_norm.shape
    n_rows_h, d = h_table.shape
    table = jnp.concatenate([h_table, w_table], axis=0)
    table = jnp.broadcast_to(table, (NW,) + table.shape).reshape(
        NW * 2 * n_rows_h, d)
    scale = jnp.full((L,), (grid_size - 1), dtype=jnp.float32)
    boxes_flat = boxes_norm.reshape(-1)
    sc = _build_sc_gather(n, n_rows_h, d)
    out = sc(boxes_flat, scale, table)
    return out.reshape(n, 2 * d, 1, 1)
